# baseline (speedup 1.0000x reference)
import functools

import jax
import jax.numpy as jnp
from jax import lax
from jax.experimental import pallas as pl
from jax.experimental.pallas import tpu as pltpu
from jax.experimental.pallas import tpu_sc as plsc

N_ROWS = 10000
D = 128
NC = 2
NS = 16
NW = NC * NS
PER = N_ROWS // NW
EXT = PER + (N_ROWS - NW * PER)
CHUNKS = (88, 80, 80, 80)


def _gather_body(w_hbm, idx_hbm, out_hbm, idx_v, rows_v, g0, g1, g2, g3, wsem):
    wid = lax.axis_index("s") * NC + lax.axis_index("c")
    base = wid * PER
    pltpu.sync_copy(idx_hbm.at[pl.ds(base, EXT)], idx_v)
    gsems = (g0, g1, g2, g3)
    gd, offs = [], []
    off = 0
    for c, n in enumerate(CHUNKS):
        gd.append(pltpu.async_copy(
            w_hbm.at[idx_v.at[pl.ds(off, n)]],
            rows_v.at[pl.ds(off, n)], gsems[c]))
        offs.append(off)
        off += n
    wd = []
    for c, n in enumerate(CHUNKS):
        gd[c].wait()
        wd.append(pltpu.async_copy(
            rows_v.at[pl.ds(offs[c], n)],
            out_hbm.at[pl.ds(base + offs[c], n)], wsem))
    for d in wd:
        d.wait()


_gather = functools.partial(
    pl.kernel,
    mesh=plsc.VectorSubcoreMesh(core_axis_name="c", subcore_axis_name="s"),
    out_type=jax.ShapeDtypeStruct((N_ROWS, D), jnp.float32),
    scratch_types=[
        pltpu.VMEM((EXT,), jnp.int32),
        pltpu.VMEM((EXT, D), jnp.float32),
        pltpu.SemaphoreType.DMA,
        pltpu.SemaphoreType.DMA,
        pltpu.SemaphoreType.DMA,
        pltpu.SemaphoreType.DMA,
        pltpu.SemaphoreType.DMA,
    ],
)(_gather_body)


def kernel(x, edge_index, edge_type, query_emb, x_batch, edge_batch, W):
    return _gather(W, x.astype(jnp.int32))


# speedup vs baseline: 1.0127x; 1.0127x over previous
import functools

import jax
import jax.numpy as jnp
from jax import lax
from jax.experimental import pallas as pl
from jax.experimental.pallas import tpu as pltpu
from jax.experimental.pallas import tpu_sc as plsc

N_ROWS = 10000
D = 128
NC = 2
NS = 16
NW = NC * NS
PER = N_ROWS // NW
EXT = PER + (N_ROWS - NW * PER)


C0 = 88
C1 = EXT - C0


def _gather_body(w_hbm, idx_hbm, out_hbm, idx_v, rows_v, g0, g1, wsem):
    wid = lax.axis_index("s") * NC + lax.axis_index("c")
    base = wid * PER
    pltpu.sync_copy(idx_hbm.at[pl.ds(base, EXT)], idx_v)
    d0 = pltpu.async_copy(w_hbm.at[idx_v.at[pl.ds(0, C0)]],
                          rows_v.at[pl.ds(0, C0)], g0)
    d1 = pltpu.async_copy(w_hbm.at[idx_v.at[pl.ds(C0, C1)]],
                          rows_v.at[pl.ds(C0, C1)], g1)
    d0.wait()
    w0 = pltpu.async_copy(rows_v.at[pl.ds(0, C0)],
                          out_hbm.at[pl.ds(base, C0)], wsem)
    d1.wait()
    pltpu.async_copy(rows_v.at[pl.ds(C0, C1)],
                     out_hbm.at[pl.ds(base + C0, C1)], wsem).wait()
    w0.wait()


_gather = functools.partial(
    pl.kernel,
    mesh=plsc.VectorSubcoreMesh(core_axis_name="c", subcore_axis_name="s"),
    out_type=jax.ShapeDtypeStruct((N_ROWS, D), jnp.float32),
    scratch_types=[
        pltpu.VMEM((EXT,), jnp.int32),
        pltpu.VMEM((EXT, D), jnp.float32),
        pltpu.SemaphoreType.DMA,
        pltpu.SemaphoreType.DMA,
        pltpu.SemaphoreType.DMA,
    ],
)(_gather_body)


def kernel(x, edge_index, edge_type, query_emb, x_batch, edge_batch, W):
    return _gather(W, x.astype(jnp.int32))


# speedup vs baseline: 1.0172x; 1.0045x over previous
import functools

import jax
import jax.numpy as jnp
from jax import lax
from jax.experimental import pallas as pl
from jax.experimental.pallas import tpu as pltpu
from jax.experimental.pallas import tpu_sc as plsc

N_ROWS = 10000
D = 128
NC = 2
NS = 16
NW = NC * NS
PER = N_ROWS // NW
EXT = PER + (N_ROWS - NW * PER)


C0 = 48
C1 = EXT - C0


def _gather_body(w_hbm, idx_hbm, out_hbm, idx_v, rows_v, g0, g1, wsem):
    wid = lax.axis_index("s") * NC + lax.axis_index("c")
    base = wid * PER
    pltpu.sync_copy(idx_hbm.at[pl.ds(base, EXT)], idx_v)
    d0 = pltpu.async_copy(w_hbm.at[idx_v.at[pl.ds(0, C0)]],
                          rows_v.at[pl.ds(0, C0)], g0)
    d1 = pltpu.async_copy(w_hbm.at[idx_v.at[pl.ds(C0, C1)]],
                          rows_v.at[pl.ds(C0, C1)], g1)
    d0.wait()
    w0 = pltpu.async_copy(rows_v.at[pl.ds(0, C0)],
                          out_hbm.at[pl.ds(base, C0)], wsem)
    d1.wait()
    pltpu.async_copy(rows_v.at[pl.ds(C0, C1)],
                     out_hbm.at[pl.ds(base + C0, C1)], wsem).wait()
    w0.wait()


_gather = functools.partial(
    pl.kernel,
    mesh=plsc.VectorSubcoreMesh(core_axis_name="c", subcore_axis_name="s"),
    out_type=jax.ShapeDtypeStruct((N_ROWS, D), jnp.float32),
    scratch_types=[
        pltpu.VMEM((EXT,), jnp.int32),
        pltpu.VMEM((EXT, D), jnp.float32),
        pltpu.SemaphoreType.DMA,
        pltpu.SemaphoreType.DMA,
        pltpu.SemaphoreType.DMA,
    ],
)(_gather_body)


def kernel(x, edge_index, edge_type, query_emb, x_batch, edge_batch, W):
    return _gather(W, x.astype(jnp.int32))
